# Initial kernel scaffold; baseline (speedup 1.0000x reference)
import jax
import jax.numpy as jnp
from jax.experimental import pallas as pl


def kernel(x, gate_w, W1, b1, W2, b2, sW1, sb1, sW2, sb2):
    raise NotImplementedError("write your pallas kernel here")


# speedup vs baseline: 1.2406x; 1.2406x over previous
import functools

import jax
import jax.numpy as jnp
from jax import lax
from jax.experimental import pallas as pl
from jax.experimental.pallas import tpu as pltpu
from jax.experimental.pallas import tpu_sc as plsc

S = 2048
D = 768
H = 3072
E = 7
EP = 128
TB = 128
NBLK = 24
CAP = NBLK * TB
NTILES = 32
CHUNK = S // NTILES
CH5 = CHUNK // 2


def _k1_body(x_ref, gw_ref, pos_ref, ce_ref, meta_ref):
    x = x_ref[...]
    gw = gw_ref[...]
    scores = jnp.dot(x, gw, preferred_element_type=jnp.float32)
    lane = lax.broadcasted_iota(jnp.int32, (S, EP), 1)
    scores = jnp.where(lane < E, scores, jnp.float32(-1e30))
    m = jnp.max(scores, axis=1, keepdims=True)
    eid = jnp.min(jnp.where(scores == m, lane, EP), axis=1, keepdims=True)
    oh = (lane == eid).astype(jnp.float32)
    counts = jnp.sum(oh, axis=0, keepdims=True)
    tr = lax.broadcasted_iota(jnp.int32, (S, S), 0)
    tc = lax.broadcasted_iota(jnp.int32, (S, S), 1)
    lstrict = (tc < tr).astype(jnp.float32)
    csum = jnp.dot(lstrict, oh, preferred_element_type=jnp.float32)
    rank = jnp.sum(csum * oh, axis=1, keepdims=True)
    counts_i = counts.astype(jnp.int32)
    lane1 = lane[:1, :]
    pc = jnp.where(lane1 < E, ((counts_i + TB) // TB) * TB, 0)
    ur = lax.broadcasted_iota(jnp.int32, (EP, EP), 0)
    uc = lax.broadcasted_iota(jnp.int32, (EP, EP), 1)
    ustrict = (ur < uc).astype(jnp.float32)
    po = jnp.dot(pc.astype(jnp.float32), ustrict,
                 preferred_element_type=jnp.float32)
    po_i = po.astype(jnp.int32)
    czpos = po_i + counts_i
    pos = (jnp.sum(oh * po, axis=1, keepdims=True) + rank).astype(jnp.int32)
    ceidx = jnp.sum(oh * czpos.astype(jnp.float32), axis=1,
                    keepdims=True).astype(jnp.int32)
    pos_ref[...] = pos
    ce_ref[...] = ceidx
    z = jnp.zeros((1, EP), jnp.int32)
    meta_ref[...] = jnp.concatenate(
        [czpos, pc, po_i, counts_i, z, z, z, z], axis=0)


def _run_k1(x2, gwp):
    return pl.pallas_call(
        _k1_body,
        out_shape=[
            jax.ShapeDtypeStruct((S, 1), jnp.int32),
            jax.ShapeDtypeStruct((S, 1), jnp.int32),
            jax.ShapeDtypeStruct((8, EP), jnp.int32),
        ],
    )(x2, gwp)


def _sc_mesh():
    return plsc.VectorSubcoreMesh(core_axis_name="c", subcore_axis_name="s")


def _k2_scatter(x2, pos, czpos):
    @functools.partial(
        pl.kernel,
        mesh=_sc_mesh(),
        out_type=jax.ShapeDtypeStruct((CAP, D), jnp.float32),
        scratch_types=[
            pltpu.VMEM((CHUNK,), jnp.int32),
            pltpu.VMEM((CHUNK, D), jnp.float32),
            pltpu.VMEM((8,), jnp.int32),
            pltpu.VMEM((8, D), jnp.float32),
            pltpu.SemaphoreType.DMA,
        ],
    )
    def k2(x_hbm, pos_hbm, cz_hbm, xs_hbm, pos_v, rows_v, cz_v, zr_v, sem):
        wid = lax.axis_index("s") * 2 + lax.axis_index("c")
        base = wid * CHUNK
        pltpu.sync_copy(pos_hbm.at[pl.ds(base, CHUNK)], pos_v)
        pltpu.sync_copy(x_hbm.at[pl.ds(base, CHUNK)], rows_v)
        pltpu.async_copy(rows_v, xs_hbm.at[pos_v], sem).wait()

        @pl.when(wid == 0)
        def _():
            for e in range(8):
                for j in range(D // 16):
                    zr_v[e, pl.ds(j * 16, 16)] = jnp.zeros((16,), jnp.float32)
            pltpu.sync_copy(cz_hbm, cz_v)
            pltpu.async_copy(zr_v, xs_hbm.at[cz_v], sem).wait()

    return k2(x2, pos, czpos)


def _k3_body(be_ref, vd_ref, xs_ref, w1_ref, b1_ref, w2_ref, b2_ref, ys_ref):
    g = pl.program_id(0)

    @pl.when(vd_ref[g] > 0)
    def _():
        xb = xs_ref[...]
        h = jnp.maximum(
            jnp.dot(xb, w1_ref[0], preferred_element_type=jnp.float32)
            + b1_ref[0], 0.0)
        ys_ref[...] = (
            jnp.dot(h, w2_ref[0], preferred_element_type=jnp.float32)
            + b2_ref[0])


def _run_k3(be, valid, xs, W1, b1, W2, b2):
    grid_spec = pltpu.PrefetchScalarGridSpec(
        num_scalar_prefetch=2,
        grid=(NBLK,),
        in_specs=[
            pl.BlockSpec((TB, D), lambda g, be, vd: (g, 0)),
            pl.BlockSpec((1, D, H), lambda g, be, vd: (be[g], 0, 0)),
            pl.BlockSpec((1, 1, H), lambda g, be, vd: (be[g], 0, 0)),
            pl.BlockSpec((1, H, D), lambda g, be, vd: (be[g], 0, 0)),
            pl.BlockSpec((1, 1, D), lambda g, be, vd: (be[g], 0, 0)),
        ],
        out_specs=pl.BlockSpec((TB, D), lambda g, be, vd: (g, 0)),
    )
    return pl.pallas_call(
        _k3_body,
        grid_spec=grid_spec,
        out_shape=jax.ShapeDtypeStruct((CAP, D), jnp.float32),
        compiler_params=pltpu.CompilerParams(
            dimension_semantics=("arbitrary",)),
    )(be, valid, xs, W1, b1.reshape(E, 1, H), W2, b2.reshape(E, 1, D))


def _k4_body(x_ref, w1_ref, b1_ref, w2_ref, b2_ref, o_ref):
    h = jnp.maximum(
        jnp.dot(x_ref[...], w1_ref[...], preferred_element_type=jnp.float32)
        + b1_ref[...], 0.0)
    o_ref[...] = (
        jnp.dot(h, w2_ref[...], preferred_element_type=jnp.float32)
        + b2_ref[...])


def _run_k4(x2, sW1, sb1, sW2, sb2):
    return pl.pallas_call(
        _k4_body,
        grid=(S // TB,),
        in_specs=[
            pl.BlockSpec((TB, D), lambda g: (g, 0)),
            pl.BlockSpec((D, H), lambda g: (0, 0)),
            pl.BlockSpec((1, H), lambda g: (0, 0)),
            pl.BlockSpec((H, D), lambda g: (0, 0)),
            pl.BlockSpec((1, D), lambda g: (0, 0)),
        ],
        out_specs=pl.BlockSpec((TB, D), lambda g: (g, 0)),
        out_shape=jax.ShapeDtypeStruct((S, D), jnp.float32),
        compiler_params=pltpu.CompilerParams(
            dimension_semantics=("arbitrary",)),
    )(x2, sW1, sb1, sW2, sb2)


def _k5_combine(shared, ys, pos, ceidx, czpos):
    @functools.partial(
        pl.kernel,
        mesh=_sc_mesh(),
        out_type=jax.ShapeDtypeStruct((S, D), jnp.float32),
        scratch_types=[
            pltpu.VMEM((CH5,), jnp.int32),
            pltpu.VMEM((CH5,), jnp.int32),
            pltpu.VMEM((8,), jnp.int32),
            pltpu.VMEM((CH5, D), jnp.float32),
            pltpu.VMEM((CH5, D), jnp.float32),
            pltpu.VMEM((CH5, D), jnp.float32),
            pltpu.VMEM((8, D), jnp.float32),
            pltpu.SemaphoreType.DMA,
            pltpu.SemaphoreType.DMA,
            pltpu.SemaphoreType.DMA,
        ],
    )
    def k5(sh_hbm, ys_hbm, pos_hbm, ce_hbm, cz_hbm, out_hbm,
           pos_v, ce_v, cz_v, sh_v, yr_v, cr_v, c8_v, sem1, sem2, sem3):
        wid = lax.axis_index("s") * 2 + lax.axis_index("c")
        pltpu.sync_copy(cz_hbm, cz_v)
        pltpu.async_copy(ys_hbm.at[cz_v], c8_v, sem3).wait()
        for j in range(D // 16):
            sl = pl.ds(j * 16, 16)
            acc = c8_v[0, sl]
            for e in range(1, E):
                acc = acc + c8_v[e, sl]
            c8_v[7, sl] = acc
        for half in range(2):
            b = wid * CHUNK + half * CH5
            pltpu.sync_copy(pos_hbm.at[pl.ds(b, CH5)], pos_v)
            pltpu.sync_copy(ce_hbm.at[pl.ds(b, CH5)], ce_v)
            pltpu.sync_copy(sh_hbm.at[pl.ds(b, CH5)], sh_v)
            h1 = pltpu.async_copy(ys_hbm.at[pos_v], yr_v, sem1)
            h2 = pltpu.async_copy(ys_hbm.at[ce_v], cr_v, sem2)
            h1.wait()
            h2.wait()

            def body(t, carry):
                for j in range(D // 16):
                    sl = pl.ds(j * 16, 16)
                    sh_v[t, sl] = (sh_v[t, sl] + yr_v[t, sl]
                                   - cr_v[t, sl] + c8_v[7, sl])
                return carry

            lax.fori_loop(0, CH5, body, 0)
            pltpu.sync_copy(sh_v, out_hbm.at[pl.ds(b, CH5)])

    return k5(shared, ys, pos, ceidx, czpos)


def kernel(x, gate_w, W1, b1, W2, b2, sW1, sb1, sW2, sb2):
    x2 = x.reshape(S, D)
    gwp = jnp.zeros((D, EP), jnp.float32).at[:, :E].set(gate_w)
    pos2, ce2, meta = _run_k1(x2, gwp)
    pos = pos2.reshape(S)
    ceidx = ce2.reshape(S)
    czpos8 = jnp.concatenate(
        [meta[0, :E], jnp.full((1,), CAP - 1, jnp.int32)])
    pc = meta[1, :E]
    po = meta[2, :E]
    gstart = jnp.arange(NBLK, dtype=jnp.int32) * TB
    ind = (gstart[None, :] >= po[:, None]) & (gstart[None, :] < (po + pc)[:, None])
    valid = jnp.any(ind, axis=0).astype(jnp.int32)
    be = jnp.where(valid > 0, jnp.argmax(ind, axis=0).astype(jnp.int32),
                   jnp.int32(E - 1))
    xs = _k2_scatter(x2, pos, czpos8)
    shared = _run_k4(x2, sW1, sb1.reshape(1, H), sW2, sb2.reshape(1, D))
    ys = _run_k3(be, valid, xs, W1, b1, W2, b2)
    out = _k5_combine(shared, ys, pos, ceidx, czpos8)
    return out.reshape(1, S, D)


# speedup vs baseline: 1.3291x; 1.0713x over previous
import functools

import jax
import jax.numpy as jnp
from jax import lax
from jax.experimental import pallas as pl
from jax.experimental.pallas import tpu as pltpu
from jax.experimental.pallas import tpu_sc as plsc

S = 2048
D = 768
H = 3072
E = 7
EP = 128
TB = 128
NBLK = 24
CAP = NBLK * TB
NTILES = 32
CHUNK = S // NTILES


def _k1_body(x_ref, gw_ref, pos_ref, eid_ref, meta_ref):
    x = x_ref[...]
    gw = gw_ref[...]
    scores = jnp.dot(x, gw, preferred_element_type=jnp.float32)
    lane = lax.broadcasted_iota(jnp.int32, (S, EP), 1)
    scores = jnp.where(lane < E, scores, jnp.float32(-1e30))
    m = jnp.max(scores, axis=1, keepdims=True)
    eid = jnp.min(jnp.where(scores == m, lane, EP), axis=1, keepdims=True)
    oh = (lane == eid).astype(jnp.float32)
    counts = jnp.sum(oh, axis=0, keepdims=True)
    tr = lax.broadcasted_iota(jnp.int32, (S, S), 0)
    tc = lax.broadcasted_iota(jnp.int32, (S, S), 1)
    lstrict = (tc < tr).astype(jnp.float32)
    csum = jnp.dot(lstrict, oh, preferred_element_type=jnp.float32)
    rank = jnp.sum(csum * oh, axis=1, keepdims=True)
    counts_i = counts.astype(jnp.int32)
    lane1 = lane[:1, :]
    pc = jnp.where(
        lane1 < E,
        jnp.maximum(((counts_i + TB - 1) // TB) * TB, TB), 0)
    ur = lax.broadcasted_iota(jnp.int32, (EP, EP), 0)
    uc = lax.broadcasted_iota(jnp.int32, (EP, EP), 1)
    ustrict = (ur < uc).astype(jnp.float32)
    po = jnp.dot(pc.astype(jnp.float32), ustrict,
                 preferred_element_type=jnp.float32)
    po_i = po.astype(jnp.int32)
    pos = (jnp.sum(oh * po, axis=1, keepdims=True) + rank).astype(jnp.int32)
    pos_ref[...] = pos
    eid_ref[...] = eid
    z = jnp.zeros((1, EP), jnp.int32)
    meta_ref[...] = jnp.concatenate(
        [pc, po_i, counts_i, z, z, z, z, z], axis=0)


def _run_k1(x2, gwp):
    return pl.pallas_call(
        _k1_body,
        out_shape=[
            jax.ShapeDtypeStruct((S, 1), jnp.int32),
            jax.ShapeDtypeStruct((S, 1), jnp.int32),
            jax.ShapeDtypeStruct((8, EP), jnp.int32),
        ],
    )(x2, gwp)


def _sc_mesh():
    return plsc.VectorSubcoreMesh(core_axis_name="c", subcore_axis_name="s")


def _k2_scatter(x2, pos):
    @functools.partial(
        pl.kernel,
        mesh=_sc_mesh(),
        out_type=jax.ShapeDtypeStruct((CAP, D), jnp.float32),
        scratch_types=[
            pltpu.VMEM((CHUNK,), jnp.int32),
            pltpu.VMEM((CHUNK, D), jnp.float32),
            pltpu.SemaphoreType.DMA,
        ],
    )
    def k2(x_hbm, pos_hbm, xs_hbm, pos_v, rows_v, sem):
        wid = lax.axis_index("s") * 2 + lax.axis_index("c")
        base = wid * CHUNK
        pltpu.sync_copy(pos_hbm.at[pl.ds(base, CHUNK)], pos_v)
        pltpu.sync_copy(x_hbm.at[pl.ds(base, CHUNK)], rows_v)
        pltpu.async_copy(rows_v, xs_hbm.at[pos_v], sem).wait()

    return k2(x2, pos)


def _k3_body(be_ref, vd_ref, fst_ref, xs_ref, w1_ref, b1_ref, w2_ref, b2_ref,
             ys_ref, ncc_ref, cmat_ref):
    g = pl.program_id(0)

    @pl.when(vd_ref[g] > 0)
    def _():
        xb = xs_ref[...]
        h = jnp.maximum(
            jnp.dot(xb, w1_ref[0], preferred_element_type=jnp.float32)
            + b1_ref[0], 0.0)
        ys_ref[...] = (
            jnp.dot(h, w2_ref[0], preferred_element_type=jnp.float32)
            + b2_ref[0])

    @pl.when(fst_ref[g] > 0)
    def _():
        hb = jnp.maximum(b1_ref[0], 0.0)
        crow = (jnp.dot(hb, w2_ref[0], preferred_element_type=jnp.float32)
                + b2_ref[0])
        e = be_ref[g]
        cmat_ref[pl.ds(e, 1), :] = crow

    @pl.when(g == NBLK - 1)
    def _():
        cm = cmat_ref[...]
        ctot = jnp.sum(cm[:E], axis=0, keepdims=True)
        ncc_ref[...] = ctot - cm


def _run_k3(be, valid, first, xs, W1, b1, W2, b2):
    grid_spec = pltpu.PrefetchScalarGridSpec(
        num_scalar_prefetch=3,
        grid=(NBLK,),
        in_specs=[
            pl.BlockSpec((TB, D), lambda g, be, vd, fs: (g, 0)),
            pl.BlockSpec((1, D, H), lambda g, be, vd, fs: (be[g], 0, 0)),
            pl.BlockSpec((1, 1, H), lambda g, be, vd, fs: (be[g], 0, 0)),
            pl.BlockSpec((1, H, D), lambda g, be, vd, fs: (be[g], 0, 0)),
            pl.BlockSpec((1, 1, D), lambda g, be, vd, fs: (be[g], 0, 0)),
        ],
        out_specs=[
            pl.BlockSpec((TB, D), lambda g, be, vd, fs: (g, 0)),
            pl.BlockSpec((8, D), lambda g, be, vd, fs: (0, 0)),
        ],
        scratch_shapes=[pltpu.VMEM((8, D), jnp.float32)],
    )
    return pl.pallas_call(
        _k3_body,
        grid_spec=grid_spec,
        out_shape=[
            jax.ShapeDtypeStruct((CAP, D), jnp.float32),
            jax.ShapeDtypeStruct((8, D), jnp.float32),
        ],
        compiler_params=pltpu.CompilerParams(
            dimension_semantics=("arbitrary",)),
    )(be, valid, first, xs, W1, b1.reshape(E, 1, H), W2, b2.reshape(E, 1, D))


def _k4_body(x_ref, w1_ref, b1_ref, w2_ref, b2_ref, rt_ref, ct_ref, o_ref):
    h = jnp.maximum(
        jnp.dot(x_ref[...], w1_ref[...], preferred_element_type=jnp.float32)
        + b1_ref[...], 0.0)
    o_ref[...] = (
        jnp.dot(h, w2_ref[...], preferred_element_type=jnp.float32)
        + b2_ref[...] + rt_ref[...] + ct_ref[...])


def _run_k4(x2, sW1, sb1, sW2, sb2, routed, cct):
    return pl.pallas_call(
        _k4_body,
        grid=(S // TB,),
        in_specs=[
            pl.BlockSpec((TB, D), lambda g: (g, 0)),
            pl.BlockSpec((D, H), lambda g: (0, 0)),
            pl.BlockSpec((1, H), lambda g: (0, 0)),
            pl.BlockSpec((H, D), lambda g: (0, 0)),
            pl.BlockSpec((1, D), lambda g: (0, 0)),
            pl.BlockSpec((TB, D), lambda g: (g, 0)),
            pl.BlockSpec((TB, D), lambda g: (g, 0)),
        ],
        out_specs=pl.BlockSpec((TB, D), lambda g: (g, 0)),
        out_shape=jax.ShapeDtypeStruct((S, D), jnp.float32),
        compiler_params=pltpu.CompilerParams(
            dimension_semantics=("arbitrary",)),
    )(x2, sW1, sb1, sW2, sb2, routed, cct)


def _k5_gather(ys, ncc, pos, eid):
    @functools.partial(
        pl.kernel,
        mesh=_sc_mesh(),
        out_type=[
            jax.ShapeDtypeStruct((S, D), jnp.float32),
            jax.ShapeDtypeStruct((S, D), jnp.float32),
        ],
        scratch_types=[
            pltpu.VMEM((CHUNK,), jnp.int32),
            pltpu.VMEM((CHUNK,), jnp.int32),
            pltpu.VMEM((CHUNK, D), jnp.float32),
            pltpu.VMEM((CHUNK, D), jnp.float32),
            pltpu.SemaphoreType.DMA,
            pltpu.SemaphoreType.DMA,
        ],
    )
    def k5(ys_hbm, ncc_hbm, pos_hbm, eid_hbm, rt_hbm, ct_hbm,
           pos_v, eid_v, yr_v, cr_v, sem1, sem2):
        wid = lax.axis_index("s") * 2 + lax.axis_index("c")
        base = wid * CHUNK
        pltpu.sync_copy(pos_hbm.at[pl.ds(base, CHUNK)], pos_v)
        pltpu.sync_copy(eid_hbm.at[pl.ds(base, CHUNK)], eid_v)
        h1 = pltpu.async_copy(ys_hbm.at[pos_v], yr_v, sem1)
        h2 = pltpu.async_copy(ncc_hbm.at[eid_v], cr_v, sem2)
        h1.wait()
        h2.wait()
        pltpu.sync_copy(yr_v, rt_hbm.at[pl.ds(base, CHUNK)])
        pltpu.sync_copy(cr_v, ct_hbm.at[pl.ds(base, CHUNK)])

    return k5(ys, ncc, pos, eid)


def kernel(x, gate_w, W1, b1, W2, b2, sW1, sb1, sW2, sb2):
    x2 = x.reshape(S, D)
    gwp = jnp.zeros((D, EP), jnp.float32).at[:, :E].set(gate_w)
    pos2, eid2, meta = _run_k1(x2, gwp)
    pos = pos2.reshape(S)
    eid = eid2.reshape(S)
    pc = meta[0, :E]
    po = meta[1, :E]
    gstart = jnp.arange(NBLK, dtype=jnp.int32) * TB
    ind = (gstart[None, :] >= po[:, None]) & (gstart[None, :] < (po + pc)[:, None])
    valid = jnp.any(ind, axis=0).astype(jnp.int32)
    be = jnp.where(valid > 0, jnp.argmax(ind, axis=0).astype(jnp.int32),
                   jnp.int32(E - 1))
    first = jnp.zeros((NBLK,), jnp.int32).at[po // TB].set(1)
    xs = _k2_scatter(x2, pos)
    ys, ncc = _run_k3(be, valid, first, xs, W1, b1, W2, b2)
    routed, cct = _k5_gather(ys, ncc, pos, eid)
    out = _run_k4(x2, sW1, sb1.reshape(1, H), sW2, sb2.reshape(1, D),
                  routed, cct)
    return out.reshape(1, S, D)


# speedup vs baseline: 1.5752x; 1.1852x over previous
import functools

import jax
import jax.numpy as jnp
from jax import lax
from jax.experimental import pallas as pl
from jax.experimental.pallas import tpu as pltpu
from jax.experimental.pallas import tpu_sc as plsc

S = 2048
D = 768
H = 3072
E = 7
EP = 128
TB = 128
NBLK = 22
CAP = NBLK * TB
NTILES = 32
CHUNK = S // NTILES


def _k1_body(x_ref, gw_ref, pos_ref, eid_ref, meta_ref):
    x = x_ref[...]
    gw = gw_ref[...]
    scores = jnp.dot(x, gw, preferred_element_type=jnp.float32)
    lane = lax.broadcasted_iota(jnp.int32, (S, EP), 1)
    scores = jnp.where(lane < E, scores, jnp.float32(-1e30))
    m = jnp.max(scores, axis=1, keepdims=True)
    eid = jnp.min(jnp.where(scores == m, lane, EP), axis=1, keepdims=True)
    oh = (lane == eid).astype(jnp.float32)
    counts = jnp.sum(oh, axis=0, keepdims=True)
    tr = lax.broadcasted_iota(jnp.int32, (S, S), 0)
    tc = lax.broadcasted_iota(jnp.int32, (S, S), 1)
    lstrict = (tc < tr).astype(jnp.float32)
    csum = jnp.dot(lstrict, oh, preferred_element_type=jnp.float32)
    rank = jnp.sum(csum * oh, axis=1, keepdims=True)
    counts_i = counts.astype(jnp.int32)
    lane1 = lane[:1, :]
    pc = jnp.where(
        lane1 < E,
        jnp.maximum(((counts_i + TB - 1) // TB) * TB, TB), 0)
    ur = lax.broadcasted_iota(jnp.int32, (EP, EP), 0)
    uc = lax.broadcasted_iota(jnp.int32, (EP, EP), 1)
    ustrict = (ur < uc).astype(jnp.float32)
    po = jnp.dot(pc.astype(jnp.float32), ustrict,
                 preferred_element_type=jnp.float32)
    po_i = po.astype(jnp.int32)
    pos = (jnp.sum(oh * po, axis=1, keepdims=True) + rank).astype(jnp.int32)
    pos_ref[...] = pos
    eid_ref[...] = eid
    z = jnp.zeros((1, EP), jnp.int32)
    meta_ref[...] = jnp.concatenate(
        [pc, po_i, counts_i, z, z, z, z, z], axis=0)


def _run_k1(x2, gwp):
    return pl.pallas_call(
        _k1_body,
        out_shape=[
            jax.ShapeDtypeStruct((S, 1), jnp.int32),
            jax.ShapeDtypeStruct((S, 1), jnp.int32),
            jax.ShapeDtypeStruct((8, EP), jnp.int32),
        ],
    )(x2, gwp)


def _sc_mesh():
    return plsc.VectorSubcoreMesh(core_axis_name="c", subcore_axis_name="s")


def _k2_scatter(x2, pos):
    @functools.partial(
        pl.kernel,
        mesh=_sc_mesh(),
        out_type=jax.ShapeDtypeStruct((CAP, D), jnp.float32),
        scratch_types=[
            pltpu.VMEM((CHUNK,), jnp.int32),
            pltpu.VMEM((CHUNK, D), jnp.float32),
            pltpu.SemaphoreType.DMA,
        ],
    )
    def k2(x_hbm, pos_hbm, xs_hbm, pos_v, rows_v, sem):
        wid = lax.axis_index("s") * 2 + lax.axis_index("c")
        base = wid * CHUNK
        pltpu.sync_copy(pos_hbm.at[pl.ds(base, CHUNK)], pos_v)
        pltpu.sync_copy(x_hbm.at[pl.ds(base, CHUNK)], rows_v)
        pltpu.async_copy(rows_v, xs_hbm.at[pos_v], sem).wait()

    return k2(x2, pos)


def _k3_body(be_ref, vd_ref, fst_ref, xs_ref, w1_ref, b1_ref, w2_ref, b2_ref,
             ys_ref, ncc_ref, cmat_ref):
    g = pl.program_id(0)

    @pl.when(g == 0)
    def _():
        cmat_ref[7:8, :] = jnp.zeros((1, D), jnp.float32)

    @pl.when(vd_ref[g] > 0)
    def _():
        xb = xs_ref[...]
        h = jnp.maximum(
            jnp.dot(xb, w1_ref[0], preferred_element_type=jnp.float32)
            + b1_ref[0], 0.0)
        ys_ref[...] = (
            jnp.dot(h, w2_ref[0], preferred_element_type=jnp.float32)
            + b2_ref[0])

    @pl.when(fst_ref[g] > 0)
    def _():
        hb = jnp.maximum(b1_ref[0], 0.0)
        crow = (jnp.dot(hb, w2_ref[0], preferred_element_type=jnp.float32)
                + b2_ref[0])
        e = be_ref[g]
        cmat_ref[pl.ds(e, 1), :] = crow

    @pl.when(g == NBLK - 1)
    def _():
        cm = cmat_ref[...]
        ctot = jnp.sum(cm[:E], axis=0, keepdims=True)
        ncc_ref[...] = ctot - cm


def _run_k3(be, valid, first, xs, W1, b1, W2, b2):
    grid_spec = pltpu.PrefetchScalarGridSpec(
        num_scalar_prefetch=3,
        grid=(NBLK,),
        in_specs=[
            pl.BlockSpec((TB, D), lambda g, be, vd, fs: (g, 0)),
            pl.BlockSpec((1, D, H), lambda g, be, vd, fs: (be[g], 0, 0)),
            pl.BlockSpec((1, 1, H), lambda g, be, vd, fs: (be[g], 0, 0)),
            pl.BlockSpec((1, H, D), lambda g, be, vd, fs: (be[g], 0, 0)),
            pl.BlockSpec((1, 1, D), lambda g, be, vd, fs: (be[g], 0, 0)),
        ],
        out_specs=[
            pl.BlockSpec((TB, D), lambda g, be, vd, fs: (g, 0)),
            pl.BlockSpec((8, D), lambda g, be, vd, fs: (0, 0)),
        ],
        scratch_shapes=[pltpu.VMEM((8, D), jnp.float32)],
    )
    return pl.pallas_call(
        _k3_body,
        grid_spec=grid_spec,
        out_shape=[
            jax.ShapeDtypeStruct((CAP, D), jnp.float32),
            jax.ShapeDtypeStruct((8, D), jnp.float32),
        ],
        compiler_params=pltpu.CompilerParams(
            dimension_semantics=("arbitrary",)),
    )(be, valid, first, xs, W1, b1.reshape(E, 1, H), W2, b2.reshape(E, 1, D))


def _k4_body(x_ref, w1_ref, b1_ref, w2_ref, b2_ref, rt_ref, ncc_ref, eid_ref,
             o_ref):
    h = jnp.maximum(
        jnp.dot(x_ref[...], w1_ref[...], preferred_element_type=jnp.float32)
        + b1_ref[...], 0.0)
    oh = (lax.broadcasted_iota(jnp.int32, (TB, 8), 1)
          == eid_ref[...]).astype(jnp.float32)
    cct = jnp.dot(oh, ncc_ref[...], preferred_element_type=jnp.float32)
    o_ref[...] = (
        jnp.dot(h, w2_ref[...], preferred_element_type=jnp.float32)
        + b2_ref[...] + rt_ref[...] + cct)


def _run_k4(x2, sW1, sb1, sW2, sb2, routed, ncc, eid2):
    return pl.pallas_call(
        _k4_body,
        grid=(S // TB,),
        in_specs=[
            pl.BlockSpec((TB, D), lambda g: (g, 0)),
            pl.BlockSpec((D, H), lambda g: (0, 0)),
            pl.BlockSpec((1, H), lambda g: (0, 0)),
            pl.BlockSpec((H, D), lambda g: (0, 0)),
            pl.BlockSpec((1, D), lambda g: (0, 0)),
            pl.BlockSpec((TB, D), lambda g: (g, 0)),
            pl.BlockSpec((8, D), lambda g: (0, 0)),
            pl.BlockSpec((TB, 1), lambda g: (g, 0)),
        ],
        out_specs=pl.BlockSpec((TB, D), lambda g: (g, 0)),
        out_shape=jax.ShapeDtypeStruct((S, D), jnp.float32),
        compiler_params=pltpu.CompilerParams(
            dimension_semantics=("arbitrary",)),
    )(x2, sW1, sb1, sW2, sb2, routed, ncc, eid2)


def _k5_gather(ys, pos):
    @functools.partial(
        pl.kernel,
        mesh=_sc_mesh(),
        out_type=jax.ShapeDtypeStruct((S, D), jnp.float32),
        scratch_types=[
            pltpu.VMEM((CHUNK,), jnp.int32),
            pltpu.VMEM((CHUNK, D), jnp.float32),
            pltpu.SemaphoreType.DMA,
        ],
    )
    def k5(ys_hbm, pos_hbm, rt_hbm, pos_v, yr_v, sem1):
        wid = lax.axis_index("s") * 2 + lax.axis_index("c")
        base = wid * CHUNK
        pltpu.sync_copy(pos_hbm.at[pl.ds(base, CHUNK)], pos_v)
        pltpu.async_copy(ys_hbm.at[pos_v], yr_v, sem1).wait()
        pltpu.sync_copy(yr_v, rt_hbm.at[pl.ds(base, CHUNK)])

    return k5(ys, pos)


def kernel(x, gate_w, W1, b1, W2, b2, sW1, sb1, sW2, sb2):
    x2 = x.reshape(S, D)
    gwp = jnp.zeros((D, EP), jnp.float32).at[:, :E].set(gate_w)
    pos2, eid2, meta = _run_k1(x2, gwp)
    pos = pos2.reshape(S)
    eid = eid2.reshape(S)
    pc = meta[0, :E]
    po = meta[1, :E]
    gstart = jnp.arange(NBLK, dtype=jnp.int32) * TB
    ind = (gstart[None, :] >= po[:, None]) & (gstart[None, :] < (po + pc)[:, None])
    valid = jnp.any(ind, axis=0).astype(jnp.int32)
    be = jnp.where(valid > 0, jnp.argmax(ind, axis=0).astype(jnp.int32),
                   jnp.int32(E - 1))
    first = jnp.zeros((NBLK,), jnp.int32).at[po // TB].set(1)
    xs = _k2_scatter(x2, pos)
    ys, ncc = _run_k3(be, valid, first, xs, W1, b1, W2, b2)
    routed = _k5_gather(ys, pos)
    out = _run_k4(x2, sW1, sb1.reshape(1, H), sW2, sb2.reshape(1, D),
                  routed, ncc, eid2)
    return out.reshape(1, S, D)


# speedup vs baseline: 1.5786x; 1.0022x over previous
import functools

import jax
import jax.numpy as jnp
from jax import lax
from jax.experimental import pallas as pl
from jax.experimental.pallas import tpu as pltpu
from jax.experimental.pallas import tpu_sc as plsc

S = 2048
D = 768
H = 3072
E = 7
EP = 128
TB = 128
NBLK = 22
CAP = NBLK * TB
NTILES = 32
CHUNK = S // NTILES


def _k1_body(x_ref, gw_ref, pos_ref, eid_ref, meta_ref):
    x = x_ref[...]
    gw = gw_ref[...]
    scores = jnp.dot(x, gw, preferred_element_type=jnp.float32)
    lane = lax.broadcasted_iota(jnp.int32, (S, EP), 1)
    scores = jnp.where(lane < E, scores, jnp.float32(-1e30))
    m = jnp.max(scores, axis=1, keepdims=True)
    eid = jnp.min(jnp.where(scores == m, lane, EP), axis=1, keepdims=True)
    oh = (lane == eid).astype(jnp.float32)
    counts = jnp.sum(oh, axis=0, keepdims=True)
    tr = lax.broadcasted_iota(jnp.int32, (S, S), 0)
    tc = lax.broadcasted_iota(jnp.int32, (S, S), 1)
    lstrict = (tc < tr).astype(jnp.float32)
    csum = jnp.dot(lstrict, oh, preferred_element_type=jnp.float32)
    rank = jnp.sum(csum * oh, axis=1, keepdims=True)
    counts_i = counts.astype(jnp.int32)
    lane1 = lane[:1, :]
    pc = jnp.where(
        lane1 < E,
        jnp.maximum(((counts_i + TB - 1) // TB) * TB, TB), 0)
    ur = lax.broadcasted_iota(jnp.int32, (EP, EP), 0)
    uc = lax.broadcasted_iota(jnp.int32, (EP, EP), 1)
    ustrict = (ur < uc).astype(jnp.float32)
    po = jnp.dot(pc.astype(jnp.float32), ustrict,
                 preferred_element_type=jnp.float32)
    po_i = po.astype(jnp.int32)
    pos = (jnp.sum(oh * po, axis=1, keepdims=True) + rank).astype(jnp.int32)
    pos_ref[...] = pos
    eid_ref[...] = eid
    z = jnp.zeros((1, EP), jnp.int32)
    meta_ref[...] = jnp.concatenate(
        [pc, po_i, counts_i, z, z, z, z, z], axis=0)


def _run_k1(x2, gwp):
    return pl.pallas_call(
        _k1_body,
        out_shape=[
            jax.ShapeDtypeStruct((S, 1), jnp.int32),
            jax.ShapeDtypeStruct((S, 1), jnp.int32),
            jax.ShapeDtypeStruct((8, EP), jnp.int32),
        ],
    )(x2, gwp)


def _sc_mesh():
    return plsc.VectorSubcoreMesh(core_axis_name="c", subcore_axis_name="s")


def _k2_scatter(x2, pos):
    @functools.partial(
        pl.kernel,
        mesh=_sc_mesh(),
        out_type=jax.ShapeDtypeStruct((CAP, D), jnp.float32),
        scratch_types=[
            pltpu.VMEM((CHUNK,), jnp.int32),
            pltpu.VMEM((CHUNK, D), jnp.float32),
            pltpu.SemaphoreType.DMA,
        ],
    )
    def k2(x_hbm, pos_hbm, xs_hbm, pos_v, rows_v, sem):
        wid = lax.axis_index("s") * 2 + lax.axis_index("c")
        base = wid * CHUNK
        pltpu.sync_copy(pos_hbm.at[pl.ds(base, CHUNK)], pos_v)
        pltpu.sync_copy(x_hbm.at[pl.ds(base, CHUNK)], rows_v)
        pltpu.async_copy(rows_v, xs_hbm.at[pos_v], sem).wait()

    return k2(x2, pos)


def _k3_body(be_ref, vd_ref, fst_ref, xs_ref, w1_ref, b1_ref, w2_ref, b2_ref,
             ys_ref, ncc_ref, cmat_ref):
    g = pl.program_id(0)

    @pl.when(g == 0)
    def _():
        cmat_ref[7:8, :] = jnp.zeros((1, D), jnp.float32)

    @pl.when(vd_ref[g] > 0)
    def _():
        xb = xs_ref[...].astype(jnp.bfloat16)
        h = jnp.maximum(
            jnp.dot(xb, w1_ref[0].astype(jnp.bfloat16),
                    preferred_element_type=jnp.float32)
            + b1_ref[0], 0.0)
        ys_ref[...] = (
            jnp.dot(h.astype(jnp.bfloat16), w2_ref[0].astype(jnp.bfloat16),
                    preferred_element_type=jnp.float32)
            + b2_ref[0])

    @pl.when(fst_ref[g] > 0)
    def _():
        hb = jnp.maximum(b1_ref[0], 0.0)
        crow = (jnp.dot(hb, w2_ref[0], preferred_element_type=jnp.float32)
                + b2_ref[0])
        e = be_ref[g]
        cmat_ref[pl.ds(e, 1), :] = crow

    @pl.when(g == NBLK - 1)
    def _():
        cm = cmat_ref[...]
        ctot = jnp.sum(cm[:E], axis=0, keepdims=True)
        ncc_ref[...] = ctot - cm


def _run_k3(be, valid, first, xs, W1, b1, W2, b2):
    grid_spec = pltpu.PrefetchScalarGridSpec(
        num_scalar_prefetch=3,
        grid=(NBLK,),
        in_specs=[
            pl.BlockSpec((TB, D), lambda g, be, vd, fs: (g, 0)),
            pl.BlockSpec((1, D, H), lambda g, be, vd, fs: (be[g], 0, 0)),
            pl.BlockSpec((1, 1, H), lambda g, be, vd, fs: (be[g], 0, 0)),
            pl.BlockSpec((1, H, D), lambda g, be, vd, fs: (be[g], 0, 0)),
            pl.BlockSpec((1, 1, D), lambda g, be, vd, fs: (be[g], 0, 0)),
        ],
        out_specs=[
            pl.BlockSpec((TB, D), lambda g, be, vd, fs: (g, 0)),
            pl.BlockSpec((8, D), lambda g, be, vd, fs: (0, 0)),
        ],
        scratch_shapes=[pltpu.VMEM((8, D), jnp.float32)],
    )
    return pl.pallas_call(
        _k3_body,
        grid_spec=grid_spec,
        out_shape=[
            jax.ShapeDtypeStruct((CAP, D), jnp.float32),
            jax.ShapeDtypeStruct((8, D), jnp.float32),
        ],
        compiler_params=pltpu.CompilerParams(
            dimension_semantics=("arbitrary",)),
    )(be, valid, first, xs, W1, b1.reshape(E, 1, H), W2, b2.reshape(E, 1, D))


def _k4_body(x_ref, w1_ref, b1_ref, w2_ref, b2_ref, rt_ref, ncc_ref, eid_ref,
             o_ref):
    h = jnp.maximum(
        jnp.dot(x_ref[...].astype(jnp.bfloat16),
                w1_ref[...].astype(jnp.bfloat16),
                preferred_element_type=jnp.float32)
        + b1_ref[...], 0.0)
    oh = (lax.broadcasted_iota(jnp.int32, (TB, 8), 1)
          == eid_ref[...]).astype(jnp.float32)
    cct = jnp.dot(oh, ncc_ref[...], preferred_element_type=jnp.float32)
    o_ref[...] = (
        jnp.dot(h.astype(jnp.bfloat16), w2_ref[...].astype(jnp.bfloat16),
                preferred_element_type=jnp.float32)
        + b2_ref[...] + rt_ref[...] + cct)


def _run_k4(x2, sW1, sb1, sW2, sb2, routed, ncc, eid2):
    return pl.pallas_call(
        _k4_body,
        grid=(S // TB,),
        in_specs=[
            pl.BlockSpec((TB, D), lambda g: (g, 0)),
            pl.BlockSpec((D, H), lambda g: (0, 0)),
            pl.BlockSpec((1, H), lambda g: (0, 0)),
            pl.BlockSpec((H, D), lambda g: (0, 0)),
            pl.BlockSpec((1, D), lambda g: (0, 0)),
            pl.BlockSpec((TB, D), lambda g: (g, 0)),
            pl.BlockSpec((8, D), lambda g: (0, 0)),
            pl.BlockSpec((TB, 1), lambda g: (g, 0)),
        ],
        out_specs=pl.BlockSpec((TB, D), lambda g: (g, 0)),
        out_shape=jax.ShapeDtypeStruct((S, D), jnp.float32),
        compiler_params=pltpu.CompilerParams(
            dimension_semantics=("arbitrary",)),
    )(x2, sW1, sb1, sW2, sb2, routed, ncc, eid2)


def _k5_gather(ys, pos):
    @functools.partial(
        pl.kernel,
        mesh=_sc_mesh(),
        out_type=jax.ShapeDtypeStruct((S, D), jnp.float32),
        scratch_types=[
            pltpu.VMEM((CHUNK,), jnp.int32),
            pltpu.VMEM((CHUNK, D), jnp.float32),
            pltpu.SemaphoreType.DMA,
        ],
    )
    def k5(ys_hbm, pos_hbm, rt_hbm, pos_v, yr_v, sem1):
        wid = lax.axis_index("s") * 2 + lax.axis_index("c")
        base = wid * CHUNK
        pltpu.sync_copy(pos_hbm.at[pl.ds(base, CHUNK)], pos_v)
        pltpu.async_copy(ys_hbm.at[pos_v], yr_v, sem1).wait()
        pltpu.sync_copy(yr_v, rt_hbm.at[pl.ds(base, CHUNK)])

    return k5(ys, pos)


def kernel(x, gate_w, W1, b1, W2, b2, sW1, sb1, sW2, sb2):
    x2 = x.reshape(S, D)
    gwp = jnp.zeros((D, EP), jnp.float32).at[:, :E].set(gate_w)
    pos2, eid2, meta = _run_k1(x2, gwp)
    pos = pos2.reshape(S)
    eid = eid2.reshape(S)
    pc = meta[0, :E]
    po = meta[1, :E]
    gstart = jnp.arange(NBLK, dtype=jnp.int32) * TB
    ind = (gstart[None, :] >= po[:, None]) & (gstart[None, :] < (po + pc)[:, None])
    valid = jnp.any(ind, axis=0).astype(jnp.int32)
    be = jnp.where(valid > 0, jnp.argmax(ind, axis=0).astype(jnp.int32),
                   jnp.int32(E - 1))
    first = jnp.zeros((NBLK,), jnp.int32).at[po // TB].set(1)
    xs = _k2_scatter(x2, pos)
    ys, ncc = _run_k3(be, valid, first, xs, W1, b1, W2, b2)
    routed = _k5_gather(ys, pos)
    out = _run_k4(x2, sW1, sb1.reshape(1, H), sW2, sb2.reshape(1, D),
                  routed, ncc, eid2)
    return out.reshape(1, S, D)


# speedup vs baseline: 1.7298x; 1.0958x over previous
import functools

import jax
import jax.numpy as jnp
from jax import lax
from jax.experimental import pallas as pl
from jax.experimental.pallas import tpu as pltpu
from jax.experimental.pallas import tpu_sc as plsc

S = 2048
D = 768
H = 3072
E = 7
EP = 128
TB = 128
NBLK = 22
CAP = NBLK * TB
NTILES = 32
CHUNK = S // NTILES


def _k1_body(x_ref, gw_ref, pos_ref, eid_ref, meta_ref):
    x = x_ref[...]
    gw = gw_ref[...]
    scores = jnp.dot(x, gw, preferred_element_type=jnp.float32)
    lane = lax.broadcasted_iota(jnp.int32, (S, EP), 1)
    scores = jnp.where(lane < E, scores, jnp.float32(-1e30))
    m = jnp.max(scores, axis=1, keepdims=True)
    eid = jnp.min(jnp.where(scores == m, lane, EP), axis=1, keepdims=True)
    oh = (lane == eid).astype(jnp.float32)
    counts = jnp.sum(oh, axis=0, keepdims=True)
    tr = lax.broadcasted_iota(jnp.int32, (S, S), 0)
    tc = lax.broadcasted_iota(jnp.int32, (S, S), 1)
    lstrict = (tc < tr).astype(jnp.float32)
    csum = jnp.dot(lstrict, oh, preferred_element_type=jnp.float32)
    rank = jnp.sum(csum * oh, axis=1, keepdims=True)
    counts_i = counts.astype(jnp.int32)
    lane1 = lane[:1, :]
    pc = jnp.where(
        lane1 < E,
        jnp.maximum(((counts_i + TB - 1) // TB) * TB, TB), 0)
    ur = lax.broadcasted_iota(jnp.int32, (EP, EP), 0)
    uc = lax.broadcasted_iota(jnp.int32, (EP, EP), 1)
    ustrict = (ur < uc).astype(jnp.float32)
    po = jnp.dot(pc.astype(jnp.float32), ustrict,
                 preferred_element_type=jnp.float32)
    po_i = po.astype(jnp.int32)
    pos = (jnp.sum(oh * po, axis=1, keepdims=True) + rank).astype(jnp.int32)
    pos_ref[...] = pos
    eid_ref[...] = eid
    z = jnp.zeros((1, EP), jnp.int32)
    meta_ref[...] = jnp.concatenate(
        [pc, po_i, counts_i, z, z, z, z, z], axis=0)


def _run_k1(x2, gwp):
    return pl.pallas_call(
        _k1_body,
        out_shape=[
            jax.ShapeDtypeStruct((S, 1), jnp.int32),
            jax.ShapeDtypeStruct((S, 1), jnp.int32),
            jax.ShapeDtypeStruct((8, EP), jnp.int32),
        ],
    )(x2, gwp)


def _sc_mesh():
    return plsc.VectorSubcoreMesh(core_axis_name="c", subcore_axis_name="s")


def _k2_scatter(x2, pos):
    @functools.partial(
        pl.kernel,
        mesh=_sc_mesh(),
        out_type=jax.ShapeDtypeStruct((CAP, D), jnp.float32),
        scratch_types=[
            pltpu.VMEM((CHUNK,), jnp.int32),
            pltpu.VMEM((CHUNK, D), jnp.float32),
            pltpu.SemaphoreType.DMA,
        ],
    )
    def k2(x_hbm, pos_hbm, xs_hbm, pos_v, rows_v, sem):
        wid = lax.axis_index("s") * 2 + lax.axis_index("c")
        base = wid * CHUNK
        pltpu.sync_copy(pos_hbm.at[pl.ds(base, CHUNK)], pos_v)
        pltpu.sync_copy(x_hbm.at[pl.ds(base, CHUNK)], rows_v)
        pltpu.async_copy(rows_v, xs_hbm.at[pos_v], sem).wait()

    return k2(x2, pos)


def _k3_body(be_ref, vd_ref, fst_ref, xs_ref, w1_hbm, b1_ref, w2_hbm, b2_ref,
             ys_ref, ncc_ref, w1buf, w2buf, sems, cmat_ref):
    g = pl.program_id(0)
    e = be_ref[g]

    def _issue(ee, slot):
        pltpu.make_async_copy(w1_hbm.at[ee], w1buf.at[slot],
                              sems.at[0, slot]).start()
        pltpu.make_async_copy(w2_hbm.at[ee], w2buf.at[slot],
                              sems.at[1, slot]).start()

    def _wait(ee, slot):
        pltpu.make_async_copy(w1_hbm.at[ee], w1buf.at[slot],
                              sems.at[0, slot]).wait()
        pltpu.make_async_copy(w2_hbm.at[ee], w2buf.at[slot],
                              sems.at[1, slot]).wait()

    @pl.when(g == 0)
    def _():
        cmat_ref[7:8, :] = jnp.zeros((1, D), jnp.float32)
        _issue(0, 0)
        _issue(1, 1)

    @pl.when((fst_ref[g] > 0) & (g > 0) & (e + 1 < E))
    def _():
        _issue(e + 1, (e + 1) % 2)

    @pl.when(fst_ref[g] > 0)
    def _():
        _wait(e, e % 2)

    def _compute(slot):
        def _():
            xb = xs_ref[...].astype(jnp.bfloat16)
            h = jnp.maximum(
                jnp.dot(xb, w1buf[slot].astype(jnp.bfloat16),
                        preferred_element_type=jnp.float32)
                + b1_ref[0], 0.0)
            ys_ref[...] = (
                jnp.dot(h.astype(jnp.bfloat16),
                        w2buf[slot].astype(jnp.bfloat16),
                        preferred_element_type=jnp.float32)
                + b2_ref[0])
        return _

    par = e % 2
    pl.when((vd_ref[g] > 0) & (par == 0))(_compute(0))
    pl.when((vd_ref[g] > 0) & (par == 1))(_compute(1))

    def _crow(slot):
        def _():
            hb = jnp.maximum(b1_ref[0], 0.0)
            crow = (jnp.dot(hb, w2buf[slot],
                            preferred_element_type=jnp.float32)
                    + b2_ref[0])
            cmat_ref[pl.ds(e, 1), :] = crow
        return _

    pl.when((fst_ref[g] > 0) & (par == 0))(_crow(0))
    pl.when((fst_ref[g] > 0) & (par == 1))(_crow(1))

    @pl.when(g == NBLK - 1)
    def _():
        cm = cmat_ref[...]
        ctot = jnp.sum(cm[:E], axis=0, keepdims=True)
        ncc_ref[...] = ctot - cm


def _run_k3(be, valid, first, xs, W1, b1, W2, b2):
    grid_spec = pltpu.PrefetchScalarGridSpec(
        num_scalar_prefetch=3,
        grid=(NBLK,),
        in_specs=[
            pl.BlockSpec((TB, D), lambda g, be, vd, fs: (g, 0)),
            pl.BlockSpec(memory_space=pltpu.MemorySpace.HBM),
            pl.BlockSpec((1, 1, H), lambda g, be, vd, fs: (be[g], 0, 0)),
            pl.BlockSpec(memory_space=pltpu.MemorySpace.HBM),
            pl.BlockSpec((1, 1, D), lambda g, be, vd, fs: (be[g], 0, 0)),
        ],
        out_specs=[
            pl.BlockSpec((TB, D), lambda g, be, vd, fs: (g, 0)),
            pl.BlockSpec((8, D), lambda g, be, vd, fs: (0, 0)),
        ],
        scratch_shapes=[
            pltpu.VMEM((2, D, H), jnp.float32),
            pltpu.VMEM((2, H, D), jnp.float32),
            pltpu.SemaphoreType.DMA((2, 2)),
            pltpu.VMEM((8, D), jnp.float32),
        ],
    )
    return pl.pallas_call(
        _k3_body,
        grid_spec=grid_spec,
        out_shape=[
            jax.ShapeDtypeStruct((CAP, D), jnp.float32),
            jax.ShapeDtypeStruct((8, D), jnp.float32),
        ],
        compiler_params=pltpu.CompilerParams(
            dimension_semantics=("arbitrary",)),
    )(be, valid, first, xs, W1, b1.reshape(E, 1, H), W2, b2.reshape(E, 1, D))


def _k4_body(x_ref, w1_ref, b1_ref, w2_ref, b2_ref, rt_ref, ncc_ref, eid_ref,
             o_ref):
    h = jnp.maximum(
        jnp.dot(x_ref[...].astype(jnp.bfloat16),
                w1_ref[...].astype(jnp.bfloat16),
                preferred_element_type=jnp.float32)
        + b1_ref[...], 0.0)
    oh = (lax.broadcasted_iota(jnp.int32, (TB, 8), 1)
          == eid_ref[...]).astype(jnp.float32)
    cct = jnp.dot(oh, ncc_ref[...], preferred_element_type=jnp.float32)
    o_ref[...] = (
        jnp.dot(h.astype(jnp.bfloat16), w2_ref[...].astype(jnp.bfloat16),
                preferred_element_type=jnp.float32)
        + b2_ref[...] + rt_ref[...] + cct)


def _run_k4(x2, sW1, sb1, sW2, sb2, routed, ncc, eid2):
    return pl.pallas_call(
        _k4_body,
        grid=(S // TB,),
        in_specs=[
            pl.BlockSpec((TB, D), lambda g: (g, 0)),
            pl.BlockSpec((D, H), lambda g: (0, 0)),
            pl.BlockSpec((1, H), lambda g: (0, 0)),
            pl.BlockSpec((H, D), lambda g: (0, 0)),
            pl.BlockSpec((1, D), lambda g: (0, 0)),
            pl.BlockSpec((TB, D), lambda g: (g, 0)),
            pl.BlockSpec((8, D), lambda g: (0, 0)),
            pl.BlockSpec((TB, 1), lambda g: (g, 0)),
        ],
        out_specs=pl.BlockSpec((TB, D), lambda g: (g, 0)),
        out_shape=jax.ShapeDtypeStruct((S, D), jnp.float32),
        compiler_params=pltpu.CompilerParams(
            dimension_semantics=("arbitrary",)),
    )(x2, sW1, sb1, sW2, sb2, routed, ncc, eid2)


def _k5_gather(ys, pos):
    @functools.partial(
        pl.kernel,
        mesh=_sc_mesh(),
        out_type=jax.ShapeDtypeStruct((S, D), jnp.float32),
        scratch_types=[
            pltpu.VMEM((CHUNK,), jnp.int32),
            pltpu.VMEM((CHUNK, D), jnp.float32),
            pltpu.SemaphoreType.DMA,
        ],
    )
    def k5(ys_hbm, pos_hbm, rt_hbm, pos_v, yr_v, sem1):
        wid = lax.axis_index("s") * 2 + lax.axis_index("c")
        base = wid * CHUNK
        pltpu.sync_copy(pos_hbm.at[pl.ds(base, CHUNK)], pos_v)
        pltpu.async_copy(ys_hbm.at[pos_v], yr_v, sem1).wait()
        pltpu.sync_copy(yr_v, rt_hbm.at[pl.ds(base, CHUNK)])

    return k5(ys, pos)


def kernel(x, gate_w, W1, b1, W2, b2, sW1, sb1, sW2, sb2):
    x2 = x.reshape(S, D)
    gwp = jnp.zeros((D, EP), jnp.float32).at[:, :E].set(gate_w)
    pos2, eid2, meta = _run_k1(x2, gwp)
    pos = pos2.reshape(S)
    eid = eid2.reshape(S)
    pc = meta[0, :E]
    po = meta[1, :E]
    gstart = jnp.arange(NBLK, dtype=jnp.int32) * TB
    ind = (gstart[None, :] >= po[:, None]) & (gstart[None, :] < (po + pc)[:, None])
    valid = jnp.any(ind, axis=0).astype(jnp.int32)
    be = jnp.where(valid > 0, jnp.argmax(ind, axis=0).astype(jnp.int32),
                   jnp.int32(E - 1))
    first = jnp.zeros((NBLK,), jnp.int32).at[po // TB].set(1)
    xs = _k2_scatter(x2, pos)
    ys, ncc = _run_k3(be, valid, first, xs, W1, b1, W2, b2)
    routed = _k5_gather(ys, pos)
    out = _run_k4(x2, sW1, sb1.reshape(1, H), sW2, sb2.reshape(1, D),
                  routed, ncc, eid2)
    return out.reshape(1, S, D)
